# baseline (speedup 1.0000x reference)
import jax
import jax.numpy as jnp
from jax import lax
from jax.experimental import pallas as pl
from jax.experimental.pallas import tpu as pltpu
from jax.experimental.pallas import tpu_sc as plsc

_V = 1000000
_D = 32
_B = 16384
_L = 16
_NW = 32
_BPW = _B // _NW


def _mf_kernel(user_ids_hbm, item_ids_hbm, user_table_hbm, item_table_hbm,
               out_hbm, uidx_v, iidx_v, urows_v, irows_v, out_v,
               sem_u, sem_i):
    wid = lax.axis_index("s") * 2 + lax.axis_index("c")
    base = wid * _BPW

    pltpu.sync_copy(user_ids_hbm.at[pl.ds(base, _BPW)], uidx_v)
    pltpu.sync_copy(item_ids_hbm.at[pl.ds(base, _BPW)], iidx_v)

    def fix(k, carry):
        u = uidx_v[pl.ds(k * _L, _L)]
        uidx_v[pl.ds(k * _L, _L)] = jnp.where((u >= 0) & (u < _V), u + 1, 0)
        i = iidx_v[pl.ds(k * _L, _L)]
        iidx_v[pl.ds(k * _L, _L)] = jnp.where((i >= 0) & (i < _V), i + 1, 0)
        return carry
    lax.fori_loop(0, _BPW // _L, fix, 0)

    cp_u = pltpu.make_async_copy(user_table_hbm.at[uidx_v], urows_v, sem_u)
    cp_i = pltpu.make_async_copy(item_table_hbm.at[iidx_v], irows_v, sem_i)
    cp_u.start()
    cp_i.start()
    cp_u.wait()
    cp_i.wait()

    lanes = lax.iota(jnp.int32, _L)

    def dot_block(rb, carry):
        rows = rb * _L + lanes

        def step(s, acc):
            d = (lanes + s) & (_D - 1)
            u = plsc.load_gather(urows_v, [rows, d])
            it = plsc.load_gather(irows_v, [rows, d])
            return acc + u * it

        acc = lax.fori_loop(0, _D, step, jnp.zeros((_L,), jnp.float32))
        out_v[pl.ds(rb * _L, _L)] = acc
        return carry

    lax.fori_loop(0, _BPW // _L, dot_block, 0)

    pltpu.sync_copy(out_v, out_hbm.at[pl.ds(base, _BPW)])


@jax.jit
def kernel(user_ids, item_ids, user_table, item_table):
    mesh = plsc.VectorSubcoreMesh(core_axis_name="c", subcore_axis_name="s")
    run = pl.kernel(
        _mf_kernel,
        out_type=jax.ShapeDtypeStruct((_B,), jnp.float32),
        mesh=mesh,
        compiler_params=pltpu.CompilerParams(
            needs_layout_passes=False, use_tc_tiling_on_sc=False),
        scratch_types=[
            pltpu.VMEM((_BPW,), jnp.int32),
            pltpu.VMEM((_BPW,), jnp.int32),
            pltpu.VMEM((_BPW, _D), jnp.float32),
            pltpu.VMEM((_BPW, _D), jnp.float32),
            pltpu.VMEM((_BPW,), jnp.float32),
            pltpu.SemaphoreType.DMA,
            pltpu.SemaphoreType.DMA,
        ],
    )
    return run(user_ids, item_ids, user_table, item_table)
